# baseline (speedup 1.0000x reference)
import jax
import jax.numpy as jnp
from jax import lax
from jax.experimental import pallas as pl
from jax.experimental.pallas import tpu as pltpu

_TOPK = 100
_NB = 125
_BS = 25
_L = 128
_C = 20


def _topk_body(cl_ref, act_ref, pb_ref, ts_ref,
               s_ref, lab_ref, q_ref, t1_ref, t2_ref, p3, bm):
    p = jax.nn.sigmoid(act_ref[0]) * jax.nn.sigmoid(cl_ref[0])
    p3[...] = p.reshape(_NB, _BS, _L)
    bm[...] = jnp.max(p3[...], axis=1)

    laneio = lax.broadcasted_iota(jnp.int32, (1, _L), 1)
    blkio = lax.broadcasted_iota(jnp.int32, (_NB, _L), 0)
    flatio = (lax.broadcasted_iota(jnp.int32, (_BS, _L), 0) * _L
              + lax.broadcasted_iota(jnp.int32, (_BS, _L), 1))
    big = jnp.int32(2 ** 30)
    ts = ts_ref[pl.program_id(0), 0]

    def body(it, carry):
        sacc, facc, t1acc, t2acc = carry
        bmv = bm[...]
        m = jnp.max(bmv)
        bstar = jnp.min(jnp.where(bmv == m, blkio, big))
        blk = p3[pl.ds(bstar, 1), :, :][0]
        fmin = jnp.min(jnp.where(blk == m, flatio, big))
        fidx = bstar * (_BS * _L) + fmin
        rin = fmin // _L
        lane = fmin % _L
        oldrow = p3[pl.ds(bstar, 1), pl.ds(rin, 1), :]
        p3[pl.ds(bstar, 1), pl.ds(rin, 1), :] = jnp.where(
            laneio[None] == lane, jnp.float32(-1.0), oldrow)
        bm[pl.ds(bstar, 1), :] = jnp.max(
            p3[pl.ds(bstar, 1), :, :], axis=1)
        qi = fidx // _C
        row = pb_ref[0, pl.ds(qi, 1), :]
        cbox = row[0, 0]
        wbox = row[0, 1]
        t1 = jnp.clip(cbox - 0.5 * wbox, 0.0, 1.0) * ts
        t2 = jnp.clip(cbox + 0.5 * wbox, 0.0, 1.0) * ts
        sel = laneio == it
        return (jnp.where(sel, m, sacc),
                jnp.where(sel, fidx, facc),
                jnp.where(sel, t1, t1acc),
                jnp.where(sel, t2, t2acc))

    init = (jnp.zeros((1, _L), jnp.float32),
            jnp.zeros((1, _L), jnp.int32),
            jnp.zeros((1, _L), jnp.float32),
            jnp.zeros((1, _L), jnp.float32))
    sacc, facc, t1acc, t2acc = lax.fori_loop(0, _TOPK, body, init)
    s_ref[0] = sacc
    lab_ref[0] = facc % _C
    q_ref[0] = facc // _C
    t1_ref[0] = t1acc
    t2_ref[0] = t2acc


def kernel(pred_boxes, class_logits, actionness_logits, target_sizes):
    B, Q, C = class_logits.shape
    N = Q * C
    R = N // _L
    cl_flat = class_logits.reshape(B, R, _L)
    act_flat = jnp.broadcast_to(
        actionness_logits, (B, Q, C)).reshape(B, R, _L)
    outs = pl.pallas_call(
        _topk_body,
        grid=(B,),
        in_specs=[pl.BlockSpec((1, R, _L), lambda b: (b, 0, 0)),
                  pl.BlockSpec((1, R, _L), lambda b: (b, 0, 0)),
                  pl.BlockSpec((1, Q, 2), lambda b: (b, 0, 0)),
                  pl.BlockSpec(memory_space=pltpu.SMEM)],
        out_specs=[pl.BlockSpec((1, 1, _L), lambda b: (b, 0, 0))] * 5,
        out_shape=[jax.ShapeDtypeStruct((B, 1, _L), jnp.float32),
                   jax.ShapeDtypeStruct((B, 1, _L), jnp.int32),
                   jax.ShapeDtypeStruct((B, 1, _L), jnp.int32),
                   jax.ShapeDtypeStruct((B, 1, _L), jnp.float32),
                   jax.ShapeDtypeStruct((B, 1, _L), jnp.float32)],
        scratch_shapes=[pltpu.VMEM((_NB, _BS, _L), jnp.float32),
                        pltpu.VMEM((_NB, _L), jnp.float32)],
    )(cl_flat, act_flat, pred_boxes, target_sizes.reshape(B, 1))
    s, lab, qidx, t1, t2 = outs
    k = _TOPK
    scores = s[:, 0, :k]
    labels = lab[:, 0, :k]
    tb_idx = qidx[:, 0, :k]
    boxes = jnp.stack([t1[:, 0, :k], t2[:, 0, :k]], axis=-1)
    return (scores, labels, boxes, tb_idx)


# speedup vs baseline: 1.0002x; 1.0002x over previous
import jax
import jax.numpy as jnp
from jax import lax
from jax.experimental import pallas as pl
from jax.experimental.pallas import tpu as pltpu

_TOPK = 100
_NB = 125
_BS = 25
_L = 128
_C = 20


def _topk_body(cl_ref, act_ref, pb_ref, ts_ref,
               s_ref, lab_ref, q_ref, t1_ref, t2_ref, p3, bm):
    p = jax.nn.sigmoid(act_ref[0]) * jax.nn.sigmoid(cl_ref[0])
    p3[...] = p.reshape(_NB, _BS, _L)
    bm[...] = jnp.max(p3[...], axis=1)

    laneio = lax.broadcasted_iota(jnp.int32, (1, _L), 1)
    blkio = lax.broadcasted_iota(jnp.int32, (_NB, _L), 0)
    flatio = (lax.broadcasted_iota(jnp.int32, (_BS, _L), 0) * _L
              + lax.broadcasted_iota(jnp.int32, (_BS, _L), 1))
    big = jnp.int32(2 ** 30)
    ts = ts_ref[pl.program_id(0), 0]

    def body(it, carry):
        sacc, facc, t1acc, t2acc = carry
        bmv = bm[...]
        m = jnp.max(bmv)
        bstar = jnp.min(jnp.where(bmv == m, blkio, big))
        blk = p3[pl.ds(bstar, 1), :, :][0]
        fmin = jnp.min(jnp.where(blk == m, flatio, big))
        fidx = bstar * (_BS * _L) + fmin
        rin = fmin // _L
        lane = fmin % _L
        oldrow = p3[pl.ds(bstar, 1), pl.ds(rin, 1), :]
        p3[pl.ds(bstar, 1), pl.ds(rin, 1), :] = jnp.where(
            laneio[None] == lane, jnp.float32(-1.0), oldrow)
        bm[pl.ds(bstar, 1), :] = jnp.max(
            p3[pl.ds(bstar, 1), :, :], axis=1)
        qi = fidx // _C
        row = pb_ref[0, pl.ds(qi, 1), :]
        cbox = row[0, 0]
        wbox = row[0, 1]
        t1 = jnp.clip(cbox - 0.5 * wbox, 0.0, 1.0) * ts
        t2 = jnp.clip(cbox + 0.5 * wbox, 0.0, 1.0) * ts
        sel = laneio == it
        return (jnp.where(sel, m, sacc),
                jnp.where(sel, fidx, facc),
                jnp.where(sel, t1, t1acc),
                jnp.where(sel, t2, t2acc))

    init = (jnp.zeros((1, _L), jnp.float32),
            jnp.zeros((1, _L), jnp.int32),
            jnp.zeros((1, _L), jnp.float32),
            jnp.zeros((1, _L), jnp.float32))
    sacc, facc, t1acc, t2acc = lax.fori_loop(0, _TOPK, body, init)
    s_ref[0] = sacc
    lab_ref[0] = facc % _C
    q_ref[0] = facc // _C
    t1_ref[0] = t1acc
    t2_ref[0] = t2acc


def kernel(pred_boxes, class_logits, actionness_logits, target_sizes):
    B, Q, C = class_logits.shape
    N = Q * C
    R = N // _L
    cl_flat = class_logits.reshape(B, R, _L)
    act_flat = jnp.broadcast_to(
        actionness_logits, (B, Q, C)).reshape(B, R, _L)
    outs = pl.pallas_call(
        _topk_body,
        grid=(B,),
        in_specs=[pl.BlockSpec((1, R, _L), lambda b: (b, 0, 0)),
                  pl.BlockSpec((1, R, _L), lambda b: (b, 0, 0)),
                  pl.BlockSpec((1, Q, 2), lambda b: (b, 0, 0)),
                  pl.BlockSpec(memory_space=pltpu.SMEM)],
        out_specs=[pl.BlockSpec((1, 1, _L), lambda b: (b, 0, 0))] * 5,
        out_shape=[jax.ShapeDtypeStruct((B, 1, _L), jnp.float32),
                   jax.ShapeDtypeStruct((B, 1, _L), jnp.int32),
                   jax.ShapeDtypeStruct((B, 1, _L), jnp.int32),
                   jax.ShapeDtypeStruct((B, 1, _L), jnp.float32),
                   jax.ShapeDtypeStruct((B, 1, _L), jnp.float32)],
        scratch_shapes=[pltpu.VMEM((_NB, _BS, _L), jnp.float32),
                        pltpu.VMEM((_NB, _L), jnp.float32)],
        compiler_params=pltpu.CompilerParams(
            dimension_semantics=("parallel",)),
    )(cl_flat, act_flat, pred_boxes, target_sizes.reshape(B, 1))
    s, lab, qidx, t1, t2 = outs
    k = _TOPK
    scores = s[:, 0, :k]
    labels = lab[:, 0, :k]
    tb_idx = qidx[:, 0, :k]
    boxes = jnp.stack([t1[:, 0, :k], t2[:, 0, :k]], axis=-1)
    return (scores, labels, boxes, tb_idx)


# speedup vs baseline: 1.0397x; 1.0395x over previous
import jax
import jax.numpy as jnp
from jax import lax
from jax.experimental import pallas as pl
from jax.experimental.pallas import tpu as pltpu

_TOPK = 100
_NB = 125
_BS = 25
_L = 128
_C = 20
_PB = 2


def _topk_body(cl_ref, act_ref, pb_ref, ts_ref,
               s_ref, lab_ref, q_ref, t1_ref, t2_ref, p3, bm):
    for i in range(_PB):
        p = jax.nn.sigmoid(act_ref[i]) * jax.nn.sigmoid(cl_ref[i])
        p3[i] = p.reshape(_NB, _BS, _L)
        bm[i] = jnp.max(p3[i], axis=1)

    laneio = lax.broadcasted_iota(jnp.int32, (1, _L), 1)
    blkio = lax.broadcasted_iota(jnp.int32, (_NB, _L), 0)
    flatio = (lax.broadcasted_iota(jnp.int32, (_BS, _L), 0) * _L
              + lax.broadcasted_iota(jnp.int32, (_BS, _L), 1))
    big = jnp.int32(2 ** 30)
    pid = pl.program_id(0)
    ts = [ts_ref[pid * _PB + i, 0] for i in range(_PB)]

    def one(i, it, acc):
        sacc, facc, t1acc, t2acc = acc
        bmv = bm[i]
        m = jnp.max(bmv)
        bstar = jnp.min(jnp.where(bmv == m, blkio, big))
        blk = p3[i, pl.ds(bstar, 1), :, :][0]
        fmin = jnp.min(jnp.where(blk == m, flatio, big))
        fidx = bstar * (_BS * _L) + fmin
        rin = fmin // _L
        lane = fmin % _L
        oldrow = p3[i, pl.ds(bstar, 1), pl.ds(rin, 1), :]
        p3[i, pl.ds(bstar, 1), pl.ds(rin, 1), :] = jnp.where(
            laneio[None] == lane, jnp.float32(-1.0), oldrow)
        bm[i, pl.ds(bstar, 1), :] = jnp.max(
            p3[i, pl.ds(bstar, 1), :, :], axis=1)
        qi = fidx // _C
        row = pb_ref[i, pl.ds(qi, 1), :]
        cbox = row[0, 0]
        wbox = row[0, 1]
        t1 = jnp.clip(cbox - 0.5 * wbox, 0.0, 1.0) * ts[i]
        t2 = jnp.clip(cbox + 0.5 * wbox, 0.0, 1.0) * ts[i]
        sel = laneio == it
        return (jnp.where(sel, m, sacc),
                jnp.where(sel, fidx, facc),
                jnp.where(sel, t1, t1acc),
                jnp.where(sel, t2, t2acc))

    def body(it, carry):
        return tuple(one(i, it, carry[i]) for i in range(_PB))

    zf = jnp.zeros((1, _L), jnp.float32)
    zi = jnp.zeros((1, _L), jnp.int32)
    init = tuple((zf, zi, zf, zf) for _ in range(_PB))
    final = lax.fori_loop(0, _TOPK, body, init)
    for i in range(_PB):
        sacc, facc, t1acc, t2acc = final[i]
        s_ref[i] = sacc
        lab_ref[i] = facc % _C
        q_ref[i] = facc // _C
        t1_ref[i] = t1acc
        t2_ref[i] = t2acc


def kernel(pred_boxes, class_logits, actionness_logits, target_sizes):
    B, Q, C = class_logits.shape
    N = Q * C
    R = N // _L
    G = B // _PB
    cl_flat = class_logits.reshape(B, R, _L)
    act_flat = jnp.broadcast_to(
        actionness_logits, (B, Q, C)).reshape(B, R, _L)
    outs = pl.pallas_call(
        _topk_body,
        grid=(G,),
        in_specs=[pl.BlockSpec((_PB, R, _L), lambda b: (b, 0, 0)),
                  pl.BlockSpec((_PB, R, _L), lambda b: (b, 0, 0)),
                  pl.BlockSpec((_PB, Q, 2), lambda b: (b, 0, 0)),
                  pl.BlockSpec(memory_space=pltpu.SMEM)],
        out_specs=[pl.BlockSpec((_PB, 1, _L), lambda b: (b, 0, 0))] * 5,
        out_shape=[jax.ShapeDtypeStruct((B, 1, _L), jnp.float32),
                   jax.ShapeDtypeStruct((B, 1, _L), jnp.int32),
                   jax.ShapeDtypeStruct((B, 1, _L), jnp.int32),
                   jax.ShapeDtypeStruct((B, 1, _L), jnp.float32),
                   jax.ShapeDtypeStruct((B, 1, _L), jnp.float32)],
        scratch_shapes=[pltpu.VMEM((_PB, _NB, _BS, _L), jnp.float32),
                        pltpu.VMEM((_PB, _NB, _L), jnp.float32)],
        compiler_params=pltpu.CompilerParams(
            dimension_semantics=("parallel",)),
    )(cl_flat, act_flat, pred_boxes, target_sizes.reshape(B, 1))
    s, lab, qidx, t1, t2 = outs
    k = _TOPK
    scores = s[:, 0, :k]
    labels = lab[:, 0, :k]
    tb_idx = qidx[:, 0, :k]
    boxes = jnp.stack([t1[:, 0, :k], t2[:, 0, :k]], axis=-1)
    return (scores, labels, boxes, tb_idx)


# speedup vs baseline: 1.0568x; 1.0164x over previous
import jax
import jax.numpy as jnp
from jax import lax
from jax.experimental import pallas as pl
from jax.experimental.pallas import tpu as pltpu

_TOPK = 100
_NB = 125
_BS = 25
_L = 128
_C = 20
_PB = 2


def _topk_body(cl_ref, act_ref, pb_ref, ts_ref,
               s_ref, lab_ref, q_ref, t1_ref, t2_ref,
               p3a, p3b, bma, bmb):
    p3 = (p3a, p3b)
    bm = (bma, bmb)
    for i in range(_PB):
        p = jax.nn.sigmoid(act_ref[i]) * jax.nn.sigmoid(cl_ref[i])
        p3[i][...] = p.reshape(_NB, _BS, _L)
        bm[i][...] = jnp.max(p3[i][...], axis=1)

    laneio = lax.broadcasted_iota(jnp.int32, (1, _L), 1)
    blkio = lax.broadcasted_iota(jnp.int32, (_NB, _L), 0)
    flatio = (lax.broadcasted_iota(jnp.int32, (_BS, _L), 0) * _L
              + lax.broadcasted_iota(jnp.int32, (_BS, _L), 1))
    big = jnp.int32(2 ** 30)
    pid = pl.program_id(0)
    ts = [ts_ref[pid * _PB + i, 0] for i in range(_PB)]

    def one(i, it, acc):
        sacc, facc, t1acc, t2acc = acc
        bmv = bm[i][...]
        m = jnp.max(bmv)
        bstar = jnp.min(jnp.where(bmv == m, blkio, big))
        blk = p3[i][pl.ds(bstar, 1), :, :][0]
        fmin = jnp.min(jnp.where(blk == m, flatio, big))
        fidx = bstar * (_BS * _L) + fmin
        rin = fmin // _L
        lane = fmin % _L
        oldrow = p3[i][pl.ds(bstar, 1), pl.ds(rin, 1), :]
        p3[i][pl.ds(bstar, 1), pl.ds(rin, 1), :] = jnp.where(
            laneio[None] == lane, jnp.float32(-1.0), oldrow)
        bm[i][pl.ds(bstar, 1), :] = jnp.max(
            p3[i][pl.ds(bstar, 1), :, :], axis=1)
        qi = fidx // _C
        row = pb_ref[i, pl.ds(qi, 1), :]
        cbox = row[0, 0]
        wbox = row[0, 1]
        t1 = jnp.clip(cbox - 0.5 * wbox, 0.0, 1.0) * ts[i]
        t2 = jnp.clip(cbox + 0.5 * wbox, 0.0, 1.0) * ts[i]
        sel = laneio == it
        return (jnp.where(sel, m, sacc),
                jnp.where(sel, fidx, facc),
                jnp.where(sel, t1, t1acc),
                jnp.where(sel, t2, t2acc))

    def body(it, carry):
        return tuple(one(i, it, carry[i]) for i in range(_PB))

    zf = jnp.zeros((1, _L), jnp.float32)
    zi = jnp.zeros((1, _L), jnp.int32)
    init = tuple((zf, zi, zf, zf) for _ in range(_PB))
    final = lax.fori_loop(0, _TOPK, body, init)
    for i in range(_PB):
        sacc, facc, t1acc, t2acc = final[i]
        s_ref[i] = sacc
        lab_ref[i] = facc % _C
        q_ref[i] = facc // _C
        t1_ref[i] = t1acc
        t2_ref[i] = t2acc


def kernel(pred_boxes, class_logits, actionness_logits, target_sizes):
    B, Q, C = class_logits.shape
    N = Q * C
    R = N // _L
    G = B // _PB
    cl_flat = class_logits.reshape(B, R, _L)
    act_flat = jnp.broadcast_to(
        actionness_logits, (B, Q, C)).reshape(B, R, _L)
    outs = pl.pallas_call(
        _topk_body,
        grid=(G,),
        in_specs=[pl.BlockSpec((_PB, R, _L), lambda b: (b, 0, 0)),
                  pl.BlockSpec((_PB, R, _L), lambda b: (b, 0, 0)),
                  pl.BlockSpec((_PB, Q, 2), lambda b: (b, 0, 0)),
                  pl.BlockSpec(memory_space=pltpu.SMEM)],
        out_specs=[pl.BlockSpec((_PB, 1, _L), lambda b: (b, 0, 0))] * 5,
        out_shape=[jax.ShapeDtypeStruct((B, 1, _L), jnp.float32),
                   jax.ShapeDtypeStruct((B, 1, _L), jnp.int32),
                   jax.ShapeDtypeStruct((B, 1, _L), jnp.int32),
                   jax.ShapeDtypeStruct((B, 1, _L), jnp.float32),
                   jax.ShapeDtypeStruct((B, 1, _L), jnp.float32)],
        scratch_shapes=[pltpu.VMEM((_NB, _BS, _L), jnp.float32),
                        pltpu.VMEM((_NB, _BS, _L), jnp.float32),
                        pltpu.VMEM((_NB, _L), jnp.float32),
                        pltpu.VMEM((_NB, _L), jnp.float32)],
        compiler_params=pltpu.CompilerParams(
            dimension_semantics=("parallel",)),
    )(cl_flat, act_flat, pred_boxes, target_sizes.reshape(B, 1))
    s, lab, qidx, t1, t2 = outs
    k = _TOPK
    scores = s[:, 0, :k]
    labels = lab[:, 0, :k]
    tb_idx = qidx[:, 0, :k]
    boxes = jnp.stack([t1[:, 0, :k], t2[:, 0, :k]], axis=-1)
    return (scores, labels, boxes, tb_idx)
